# baseline (speedup 1.0000x reference)
import functools

import jax
import jax.numpy as jnp
from jax import lax
from jax.experimental import pallas as pl
from jax.experimental.pallas import tpu as pltpu
from jax.experimental.pallas import tpu_sc as plsc

_R = 2048
_NC, _NS = 2, 16
_NW = _NC * _NS


def _lnc_relu(hc, g, be):
    v = jnp.mean(hc * hc, axis=-1, keepdims=True)
    return jnp.maximum(hc * jax.lax.rsqrt(v + 1e-5) * g + be, 0.0)


def _center(W, b):
    return W - jnp.mean(W, axis=1, keepdims=True), b - jnp.mean(b)


def _sc_gather_call(table_i32, idx_flat):
    N = idx_flat.shape[0]
    npw = N // _NW
    mesh = plsc.VectorSubcoreMesh(core_axis_name="c", subcore_axis_name="s")

    @functools.partial(
        pl.kernel,
        mesh=mesh,
        compiler_params=pltpu.CompilerParams(use_tc_tiling_on_sc=False),
        out_type=jax.ShapeDtypeStruct((N, 16), jnp.int32),
        scratch_types=[
            pltpu.VMEM((npw,), jnp.int32),
            pltpu.VMEM((npw, 16), jnp.int32),
            pltpu.SemaphoreType.DMA,
        ],
    )
    def gather_k(table_hbm, idx_hbm, out_hbm, idx_v, rows_v, sem):
        wid = lax.axis_index("s") * _NC + lax.axis_index("c")
        base = wid * npw
        pltpu.sync_copy(idx_hbm.at[pl.ds(base, npw)], idx_v)
        pltpu.async_copy(table_hbm.at[idx_v], rows_v, sem).wait()
        pltpu.sync_copy(rows_v, out_hbm.at[pl.ds(base, npw)])

    return gather_k(table_i32, idx_flat)


def _mlp_body(emb_ref, role_ref, w1a_ref, w1b_ref, b1_ref, g1_ref, be1_ref,
              w2_ref, b2_ref, g2_ref, be2_ref, w3_ref, b3_ref, g3_ref,
              be3_ref, w4t_ref, b4_ref, out_ref):
    h = jnp.dot(emb_ref[...], w1a_ref[...], preferred_element_type=jnp.float32)
    h = h + jnp.dot(role_ref[...], w1b_ref[...],
                    preferred_element_type=jnp.float32)
    h = _lnc_relu(h + b1_ref[...], g1_ref[...], be1_ref[...])
    h = jnp.dot(h.astype(jnp.bfloat16), w2_ref[...],
                preferred_element_type=jnp.float32)
    h = _lnc_relu(h + b2_ref[...], g2_ref[...], be2_ref[...])
    h = jnp.dot(h.astype(jnp.bfloat16), w3_ref[...],
                preferred_element_type=jnp.float32)
    h = _lnc_relu(h + b3_ref[...], g3_ref[...], be3_ref[...])
    o = jnp.sum(h * w4t_ref[...], axis=1, keepdims=True) + b4_ref[...]
    out_ref[...] = jax.nn.sigmoid(o)


def kernel(champion_ids, role, embed, W1, b1, g1, be1, W2, b2, g2, be2, W3,
           b3, g3, be3, W4, b4):
    B = champion_ids.shape[0]
    table = embed.at[0].set(0.0).astype(jnp.bfloat16)
    table_i32 = jax.lax.bitcast_convert_type(
        table.reshape(166, 16, 2), jnp.int32)
    idx_flat = champion_ids.reshape(-1)
    emb_i32 = _sc_gather_call(table_i32, idx_flat)
    emb = jax.lax.bitcast_convert_type(emb_i32, jnp.bfloat16)
    emb = emb.reshape(B, 352)

    W1c, b1c = _center(W1, b1)
    W2c, b2c = _center(W2, b2)
    W3c, b3c = _center(W3, b3)
    w1a = W1c[:352].astype(jnp.bfloat16)
    w1b = W1c[352:].astype(jnp.bfloat16)
    W2b = W2c.astype(jnp.bfloat16)
    W3b = W3c.astype(jnp.bfloat16)
    role_b = role.astype(jnp.bfloat16)
    row = lambda v: v.reshape(1, -1)
    grid = B // _R
    tile = lambda i: (i, 0)
    rep = lambda i: (0, 0)
    out = pl.pallas_call(
        _mlp_body,
        grid=(grid,),
        in_specs=[
            pl.BlockSpec((_R, 352), tile),
            pl.BlockSpec((_R, 5), tile),
        ] + [pl.BlockSpec(w.shape, rep) for w in (
            w1a, w1b, row(b1c), row(g1), row(be1), W2b, row(b2c), row(g2),
            row(be2), W3b, row(b3c), row(g3), row(be3), W4.reshape(1, -1),
            row(b4))],
        out_specs=pl.BlockSpec((_R, 1), tile),
        out_shape=jax.ShapeDtypeStruct((B, 1), jnp.float32),
    )(emb, role_b, w1a, w1b, row(b1c), row(g1), row(be1), W2b, row(b2c),
      row(g2), row(be2), W3b, row(b3c), row(g3), row(be3), W4.reshape(1, -1),
      row(b4))
    return out[:, 0]


# speedup vs baseline: 15.8585x; 15.8585x over previous
import functools

import jax
import jax.numpy as jnp
from jax import lax
from jax.experimental import pallas as pl
from jax.experimental.pallas import tpu as pltpu
from jax.experimental.pallas import tpu_sc as plsc

_R = 2048
_NC, _NS = 2, 16
_NW = _NC * _NS
_NSLOT = 11
_PW = 16


def _lnc_relu(hc, g, be):
    v = jnp.mean(hc * hc, axis=-1, keepdims=True)
    return jnp.maximum(hc * jax.lax.rsqrt(v + 1e-5) * g + be, 0.0)


def _center(W, b):
    return W - jnp.mean(W, axis=1, keepdims=True), b - jnp.mean(b)


def _sc_gather_call(table_t, idx_t, B):
    spw = B // _NW
    chunk = 256
    mesh = plsc.VectorSubcoreMesh(core_axis_name="c", subcore_axis_name="s")

    @functools.partial(
        pl.kernel,
        mesh=mesh,
        compiler_params=pltpu.CompilerParams(use_tc_tiling_on_sc=False,
                                             needs_layout_passes=False),
        out_type=jax.ShapeDtypeStruct((B, 16 * _PW), jnp.int32),
        scratch_types=[
            pltpu.VMEM((16, 256), jnp.int32),
            pltpu.VMEM((_NSLOT, spw), jnp.int32),
            pltpu.VMEM((chunk, 16 * _PW), jnp.int32),
        ],
    )
    def gather_k(table_hbm, idx_hbm, out_hbm, tab_v, idx_v, buf_v):
        wid = lax.axis_index("s") * _NC + lax.axis_index("c")
        iota = lax.broadcasted_iota(jnp.int32, (16,), 0)
        pltpu.sync_copy(table_hbm, tab_v)
        pltpu.sync_copy(
            idx_hbm.at[pl.ds(0, _NSLOT), pl.ds(wid * spw, spw)], idx_v)

        def make_group_body(buf_base):
            def group_body(g, _):
                rows = jnp.full((16,), g * 16 - buf_base, jnp.int32) + iota
                for j in range(_NSLOT):
                    ids16 = idx_v[j, pl.ds(g * 16, 16)]
                    for c in range(_PW):
                        vals = plsc.load_gather(
                            tab_v, [jnp.full((16,), c, jnp.int32), ids16])
                        plsc.store_scatter(
                            buf_v,
                            [rows, jnp.full((16,), j * _PW + c, jnp.int32)],
                            vals)
                return 0
            return group_body

        for ck in range(spw // chunk):
            lax.fori_loop(ck * (chunk // 16), (ck + 1) * (chunk // 16),
                          make_group_body(ck * chunk), 0, unroll=False)
            pltpu.sync_copy(
                buf_v, out_hbm.at[pl.ds(wid * spw + ck * chunk, chunk)])

    return gather_k(table_t, idx_t)


def _mlp_body(emb_ref, role_ref, w1e_ref, w1o_ref, w1b_ref, b1_ref, g1_ref,
              be1_ref, w2_ref, b2_ref, g2_ref, be2_ref, w3_ref, b3_ref,
              g3_ref, be3_ref, w4t_ref, b4_ref, out_ref):
    raw = emb_ref[...]
    lo = jax.lax.bitcast_convert_type(raw << 16, jnp.float32)
    hi = jax.lax.bitcast_convert_type(raw & jnp.int32(-65536), jnp.float32)
    elo = lo[:, :176].astype(jnp.bfloat16)
    ehi = hi[:, :176].astype(jnp.bfloat16)
    h = jnp.dot(elo, w1e_ref[...], preferred_element_type=jnp.float32)
    h = h + jnp.dot(ehi, w1o_ref[...], preferred_element_type=jnp.float32)
    h = h + jnp.dot(role_ref[...], w1b_ref[...],
                    preferred_element_type=jnp.float32)
    h = _lnc_relu(h + b1_ref[...], g1_ref[...], be1_ref[...])
    h = jnp.dot(h.astype(jnp.bfloat16), w2_ref[...],
                preferred_element_type=jnp.float32)
    h = _lnc_relu(h + b2_ref[...], g2_ref[...], be2_ref[...])
    h = jnp.dot(h.astype(jnp.bfloat16), w3_ref[...],
                preferred_element_type=jnp.float32)
    h = _lnc_relu(h + b3_ref[...], g3_ref[...], be3_ref[...])
    o = jnp.sum(h * w4t_ref[...], axis=1, keepdims=True) + b4_ref[...]
    out_ref[...] = jax.nn.sigmoid(o)


def kernel(champion_ids, role, embed, W1, b1, g1, be1, W2, b2, g2, be2, W3,
           b3, g3, be3, W4, b4):
    B = champion_ids.shape[0]
    table = embed.at[0].set(0.0).astype(jnp.bfloat16)
    table_i32 = jax.lax.bitcast_convert_type(
        table.reshape(166, _PW, 2), jnp.int32)
    table_t = jnp.zeros((16, 256), jnp.int32).at[:, :166].set(table_i32.T)
    idx_t = jnp.zeros((16, B), jnp.int32).at[:_NSLOT].set(champion_ids.T)

    emb_i32 = _sc_gather_call(table_t, idx_t, B)

    W1c, b1c = _center(W1, b1)
    W2c, b2c = _center(W2, b2)
    W3c, b3c = _center(W3, b3)
    w1e = W1c[0:352:2].astype(jnp.bfloat16)
    w1o = W1c[1:352:2].astype(jnp.bfloat16)
    w1b = W1c[352:].astype(jnp.bfloat16)
    W2b = W2c.astype(jnp.bfloat16)
    W3b = W3c.astype(jnp.bfloat16)
    role_b = role.astype(jnp.bfloat16)
    row = lambda v: v.reshape(1, -1)
    grid = B // _R
    tile = lambda i: (i, 0)
    rep = lambda i: (0, 0)
    out = pl.pallas_call(
        _mlp_body,
        grid=(grid,),
        in_specs=[
            pl.BlockSpec((_R, 256), tile),
            pl.BlockSpec((_R, 5), tile),
        ] + [pl.BlockSpec(w.shape, rep) for w in (
            w1e, w1o, w1b, row(b1c), row(g1), row(be1), W2b, row(b2c),
            row(g2), row(be2), W3b, row(b3c), row(g3), row(be3),
            W4.reshape(1, -1), row(b4))],
        out_specs=pl.BlockSpec((_R, 1), tile),
        out_shape=jax.ShapeDtypeStruct((B, 1), jnp.float32),
    )(emb_i32, role_b, w1e, w1o, w1b, row(b1c), row(g1), row(be1), W2b,
      row(b2c), row(g2), row(be2), W3b, row(b3c), row(g3), row(be3),
      W4.reshape(1, -1), row(b4))
    return out[:, 0]


# speedup vs baseline: 17.5620x; 1.1074x over previous
import functools

import jax
import jax.numpy as jnp
from jax import lax
from jax.experimental import pallas as pl
from jax.experimental.pallas import tpu as pltpu
from jax.experimental.pallas import tpu_sc as plsc

_R = 2048
_NC, _NS = 2, 16
_NW = _NC * _NS
_NSLOT = 11
_PW = 16


def _lnc_relu(hc, g, be):
    v = jnp.mean(hc * hc, axis=-1, keepdims=True)
    return jnp.maximum(hc * jax.lax.rsqrt(v + 1e-5) * g + be, 0.0)


def _center(W, b):
    return W - jnp.mean(W, axis=1, keepdims=True), b - jnp.mean(b)


def _sc_gather_call(table_t, idx_t, B):
    spw = B // _NW
    chunk = 256
    mesh = plsc.VectorSubcoreMesh(core_axis_name="c", subcore_axis_name="s")

    @functools.partial(
        pl.kernel,
        mesh=mesh,
        compiler_params=pltpu.CompilerParams(use_tc_tiling_on_sc=False,
                                             needs_layout_passes=False),
        out_type=jax.ShapeDtypeStruct((B, 16 * _PW), jnp.int32),
        scratch_types=[
            pltpu.VMEM((16, 256), jnp.int32),
            pltpu.VMEM((_NSLOT, spw), jnp.int32),
            pltpu.VMEM((chunk, 16 * _PW), jnp.int32),
        ],
    )
    def gather_k(table_hbm, idx_hbm, out_hbm, tab_v, idx_v, buf_v):
        wid = lax.axis_index("s") * _NC + lax.axis_index("c")
        iota = lax.broadcasted_iota(jnp.int32, (16,), 0)
        pltpu.sync_copy(table_hbm, tab_v)
        pltpu.sync_copy(
            idx_hbm.at[pl.ds(0, _NSLOT), pl.ds(wid * spw, spw)], idx_v)

        for ck in range(spw // chunk):
            @plsc.parallel_loop(ck * (chunk // 16), (ck + 1) * (chunk // 16))
            def group_body(g, _ck=ck):
                rows = jnp.full((16,), g * 16 - _ck * chunk, jnp.int32) + iota
                for j in range(_NSLOT):
                    ids16 = idx_v[j, pl.ds(g * 16, 16)]
                    for c in range(_PW):
                        vals = plsc.load_gather(
                            tab_v, [jnp.full((16,), c, jnp.int32), ids16])
                        plsc.store_scatter(
                            buf_v,
                            [rows, jnp.full((16,), j * _PW + c, jnp.int32)],
                            vals)

            pltpu.sync_copy(
                buf_v, out_hbm.at[pl.ds(wid * spw + ck * chunk, chunk)])

    return gather_k(table_t, idx_t)


def _mlp_body(emb_ref, role_ref, w1e_ref, w1o_ref, w1b_ref, b1_ref, g1_ref,
              be1_ref, w2_ref, b2_ref, g2_ref, be2_ref, w3_ref, b3_ref,
              g3_ref, be3_ref, w4t_ref, b4_ref, out_ref):
    raw = emb_ref[...]
    lo = jax.lax.bitcast_convert_type(raw << 16, jnp.float32)
    hi = jax.lax.bitcast_convert_type(raw & jnp.int32(-65536), jnp.float32)
    elo = lo[:, :176].astype(jnp.bfloat16)
    ehi = hi[:, :176].astype(jnp.bfloat16)
    h = jnp.dot(elo, w1e_ref[...], preferred_element_type=jnp.float32)
    h = h + jnp.dot(ehi, w1o_ref[...], preferred_element_type=jnp.float32)
    h = h + jnp.dot(role_ref[...], w1b_ref[...],
                    preferred_element_type=jnp.float32)
    h = _lnc_relu(h + b1_ref[...], g1_ref[...], be1_ref[...])
    h = jnp.dot(h.astype(jnp.bfloat16), w2_ref[...],
                preferred_element_type=jnp.float32)
    h = _lnc_relu(h + b2_ref[...], g2_ref[...], be2_ref[...])
    h = jnp.dot(h.astype(jnp.bfloat16), w3_ref[...],
                preferred_element_type=jnp.float32)
    h = _lnc_relu(h + b3_ref[...], g3_ref[...], be3_ref[...])
    o = jnp.sum(h * w4t_ref[...], axis=1, keepdims=True) + b4_ref[...]
    out_ref[...] = jax.nn.sigmoid(o)


def kernel(champion_ids, role, embed, W1, b1, g1, be1, W2, b2, g2, be2, W3,
           b3, g3, be3, W4, b4):
    B = champion_ids.shape[0]
    table = embed.at[0].set(0.0).astype(jnp.bfloat16)
    table_i32 = jax.lax.bitcast_convert_type(
        table.reshape(166, _PW, 2), jnp.int32)
    table_t = jnp.zeros((16, 256), jnp.int32).at[:, :166].set(table_i32.T)
    idx_t = jnp.zeros((16, B), jnp.int32).at[:_NSLOT].set(champion_ids.T)

    emb_i32 = _sc_gather_call(table_t, idx_t, B)

    W1c, b1c = _center(W1, b1)
    W2c, b2c = _center(W2, b2)
    W3c, b3c = _center(W3, b3)
    w1e = W1c[0:352:2].astype(jnp.bfloat16)
    w1o = W1c[1:352:2].astype(jnp.bfloat16)
    w1b = W1c[352:].astype(jnp.bfloat16)
    W2b = W2c.astype(jnp.bfloat16)
    W3b = W3c.astype(jnp.bfloat16)
    role_b = role.astype(jnp.bfloat16)
    row = lambda v: v.reshape(1, -1)
    grid = B // _R
    tile = lambda i: (i, 0)
    rep = lambda i: (0, 0)
    out = pl.pallas_call(
        _mlp_body,
        grid=(grid,),
        in_specs=[
            pl.BlockSpec((_R, 256), tile),
            pl.BlockSpec((_R, 5), tile),
        ] + [pl.BlockSpec(w.shape, rep) for w in (
            w1e, w1o, w1b, row(b1c), row(g1), row(be1), W2b, row(b2c),
            row(g2), row(be2), W3b, row(b3c), row(g3), row(be3),
            W4.reshape(1, -1), row(b4))],
        out_specs=pl.BlockSpec((_R, 1), tile),
        out_shape=jax.ShapeDtypeStruct((B, 1), jnp.float32),
    )(emb_i32, role_b, w1e, w1o, w1b, row(b1c), row(g1), row(be1), W2b,
      row(b2c), row(g2), row(be2), W3b, row(b3c), row(g3), row(be3),
      W4.reshape(1, -1), row(b4))
    return out[:, 0]


# speedup vs baseline: 19.5363x; 1.1124x over previous
import functools

import jax
import jax.numpy as jnp
from jax import lax
from jax.experimental import pallas as pl
from jax.experimental.pallas import tpu as pltpu
from jax.experimental.pallas import tpu_sc as plsc

_R = 2048
_NC, _NS = 2, 16
_NW = _NC * _NS
_NSLOT = 11
_PW = 16


def _lnc_relu(hc, g, be):
    v = jnp.mean(hc * hc, axis=-1, keepdims=True)
    return jnp.maximum(hc * jax.lax.rsqrt(v + 1e-5) * g + be, 0.0)


def _center(W, b):
    return W - jnp.mean(W, axis=1, keepdims=True), b - jnp.mean(b)


def _sc_gather_call(table_t, idx2d, B):
    spw = B // _NW
    chunk = 256
    ipw = spw * _NSLOT // 128
    mesh = plsc.VectorSubcoreMesh(core_axis_name="c", subcore_axis_name="s")

    @functools.partial(
        pl.kernel,
        mesh=mesh,
        compiler_params=pltpu.CompilerParams(use_tc_tiling_on_sc=False,
                                             needs_layout_passes=False),
        out_type=jax.ShapeDtypeStruct((B, 16 * _PW), jnp.int32),
        scratch_types=[
            pltpu.VMEM((16, 256), jnp.int32),
            pltpu.VMEM((ipw, 128), jnp.int32),
            pltpu.VMEM((chunk, 16 * _PW + 1), jnp.int32),
        ],
    )
    def gather_k(table_hbm, idx_hbm, out_hbm, tab_v, idx_v, buf_v):
        wid = lax.axis_index("s") * _NC + lax.axis_index("c")
        iota = lax.broadcasted_iota(jnp.int32, (16,), 0)
        pltpu.sync_copy(table_hbm, tab_v)
        pltpu.sync_copy(idx_hbm.at[pl.ds(wid * ipw, ipw)], idx_v)

        def chunk_body(ck, _):
            @plsc.parallel_loop(ck * (chunk // 16), (ck + 1) * (chunk // 16))
            def group_body(g):
                brows = jnp.full((16,), (g - ck * (chunk // 16)) * 16,
                                 jnp.int32) + iota
                svec = jnp.full((16,), g * 16 * _NSLOT, jnp.int32) \
                    + iota * _NSLOT
                for j in range(_NSLOT):
                    flats = svec + j
                    ids16 = plsc.load_gather(
                        idx_v, [flats >> 7, flats & 127])
                    for c in range(_PW):
                        vals = plsc.load_gather(
                            tab_v, [jnp.full((16,), c, jnp.int32), ids16])
                        plsc.store_scatter(
                            buf_v,
                            [brows, jnp.full((16,), j * _PW + c, jnp.int32)],
                            vals)

            pltpu.sync_copy(
                buf_v.at[pl.ds(0, chunk), pl.ds(0, 16 * _PW)],
                out_hbm.at[pl.ds(wid * spw + ck * chunk, chunk)])
            return 0

        lax.fori_loop(0, spw // chunk, chunk_body, 0, unroll=False)

    return gather_k(table_t, idx2d)


def _mlp_body(emb_ref, role_ref, w1e_ref, w1o_ref, w1b_ref, b1_ref, g1_ref,
              be1_ref, w2_ref, b2_ref, g2_ref, be2_ref, w3_ref, b3_ref,
              g3_ref, be3_ref, w4t_ref, b4_ref, out_ref):
    raw = emb_ref[...]
    lo = jax.lax.bitcast_convert_type(raw << 16, jnp.float32)
    hi = jax.lax.bitcast_convert_type(raw & jnp.int32(-65536), jnp.float32)
    elo = lo[:, :176].astype(jnp.bfloat16)
    ehi = hi[:, :176].astype(jnp.bfloat16)
    h = jnp.dot(elo, w1e_ref[...], preferred_element_type=jnp.float32)
    h = h + jnp.dot(ehi, w1o_ref[...], preferred_element_type=jnp.float32)
    h = h + jnp.dot(role_ref[...], w1b_ref[...],
                    preferred_element_type=jnp.float32)
    h = _lnc_relu(h + b1_ref[...], g1_ref[...], be1_ref[...])
    h = jnp.dot(h.astype(jnp.bfloat16), w2_ref[...],
                preferred_element_type=jnp.float32)
    h = _lnc_relu(h + b2_ref[...], g2_ref[...], be2_ref[...])
    h = jnp.dot(h.astype(jnp.bfloat16), w3_ref[...],
                preferred_element_type=jnp.float32)
    h = _lnc_relu(h + b3_ref[...], g3_ref[...], be3_ref[...])
    o = jnp.sum(h * w4t_ref[...], axis=1, keepdims=True) + b4_ref[...]
    out_ref[...] = jax.nn.sigmoid(o)


def kernel(champion_ids, role, embed, W1, b1, g1, be1, W2, b2, g2, be2, W3,
           b3, g3, be3, W4, b4):
    B = champion_ids.shape[0]
    table = embed.at[0].set(0.0).astype(jnp.bfloat16)
    table_i32 = jax.lax.bitcast_convert_type(
        table.reshape(166, _PW, 2), jnp.int32)
    table_t = jnp.zeros((16, 256), jnp.int32).at[:, :166].set(table_i32.T)
    idx2d = champion_ids.reshape(B * _NSLOT // 128, 128)

    emb_i32 = _sc_gather_call(table_t, idx2d, B)

    W1c, b1c = _center(W1, b1)
    W2c, b2c = _center(W2, b2)
    W3c, b3c = _center(W3, b3)
    w1e = W1c[0:352:2].astype(jnp.bfloat16)
    w1o = W1c[1:352:2].astype(jnp.bfloat16)
    w1b = W1c[352:].astype(jnp.bfloat16)
    W2b = W2c.astype(jnp.bfloat16)
    W3b = W3c.astype(jnp.bfloat16)
    role_b = role.astype(jnp.bfloat16)
    row = lambda v: v.reshape(1, -1)
    grid = B // _R
    tile = lambda i: (i, 0)
    rep = lambda i: (0, 0)
    out = pl.pallas_call(
        _mlp_body,
        grid=(grid,),
        in_specs=[
            pl.BlockSpec((_R, 256), tile),
            pl.BlockSpec((_R, 5), tile),
        ] + [pl.BlockSpec(w.shape, rep) for w in (
            w1e, w1o, w1b, row(b1c), row(g1), row(be1), W2b, row(b2c),
            row(g2), row(be2), W3b, row(b3c), row(g3), row(be3),
            W4.reshape(1, -1), row(b4))],
        out_specs=pl.BlockSpec((_R, 1), tile),
        out_shape=jax.ShapeDtypeStruct((B, 1), jnp.float32),
    )(emb_i32, role_b, w1e, w1o, w1b, row(b1c), row(g1), row(be1), W2b,
      row(b2c), row(g2), row(be2), W3b, row(b3c), row(g3), row(be3),
      W4.reshape(1, -1), row(b4))
    return out[:, 0]
